# Initial kernel scaffold; baseline (speedup 1.0000x reference)
import jax
import jax.numpy as jnp
from jax.experimental import pallas as pl


def kernel(x, w_ih_fused, w_hh_blk, b_fused):
    raise NotImplementedError("write your pallas kernel here")


# speedup vs baseline: 1.1462x; 1.1462x over previous
import functools

import jax
import jax.numpy as jnp
from jax import lax
from jax.experimental import pallas as pl
from jax.experimental.pallas import tpu as pltpu


def _bilstm_body(x_ref, wih_ref, whh_ref, b_ref, out_ref, *, seq_len, nb, hidden):
    T, Nb, H = seq_len, nb, hidden
    HH = 2 * H
    G = 8 * H

    x = x_ref[...].reshape(T * Nb, x_ref.shape[-1])
    wih = wih_ref[...]
    whh = whh_ref[...]
    b = b_ref[...]

    pre = jnp.dot(x, wih, preferred_element_type=jnp.float32) + b

    lane = lax.broadcasted_iota(jnp.int32, (Nb, G), 1)
    is_fwd = (lane % HH) < H

    h = jnp.zeros((Nb, HH), jnp.float32)
    c = jnp.zeros((Nb, HH), jnp.float32)

    for t in range(T):
        tb = T - 1 - t
        pre_t = jnp.where(is_fwd,
                          pre[t * Nb:(t + 1) * Nb, :],
                          pre[tb * Nb:(tb + 1) * Nb, :])
        gates = pre_t + jnp.dot(h.astype(jnp.bfloat16), whh,
                                preferred_element_type=jnp.float32)
        s_if = jax.nn.sigmoid(gates[:, 0:2 * HH])
        g_g = jnp.tanh(gates[:, 2 * HH:3 * HH])
        o_g = jax.nn.sigmoid(gates[:, 3 * HH:4 * HH])
        c = s_if[:, HH:2 * HH] * c + s_if[:, 0:HH] * g_g
        h = o_g * jnp.tanh(c)
        out_ref[t, :, 0:H] = h[:, 0:H].astype(out_ref.dtype)
        out_ref[tb, :, H:2 * H] = h[:, H:2 * H].astype(out_ref.dtype)


@jax.jit
def kernel(x, w_ih_fused, w_hh_blk, b_fused):
    N, T, E = x.shape
    H = w_hh_blk.shape[0] // 2
    NC = 2
    Nb = N // NC

    xt = jnp.transpose(x, (1, 0, 2)).astype(jnp.bfloat16)
    wih = w_ih_fused.astype(jnp.bfloat16)
    whh = w_hh_blk.astype(jnp.bfloat16)

    body = functools.partial(_bilstm_body, seq_len=T, nb=Nb, hidden=H)
    out = pl.pallas_call(
        body,
        out_shape=jax.ShapeDtypeStruct((T, N, 2 * H), x.dtype),
        grid=(NC,),
        in_specs=[
            pl.BlockSpec((T, Nb, E), lambda i: (0, i, 0)),
            pl.BlockSpec((E, 8 * H), lambda i: (0, 0)),
            pl.BlockSpec((2 * H, 8 * H), lambda i: (0, 0)),
            pl.BlockSpec((1, 8 * H), lambda i: (0, 0)),
        ],
        out_specs=pl.BlockSpec((T, Nb, 2 * H), lambda i: (0, i, 0)),
        compiler_params=pltpu.CompilerParams(
            dimension_semantics=("parallel",)),
    )(xt, wih, whh, b_fused)

    return jnp.transpose(out, (1, 0, 2))


# speedup vs baseline: 1.8496x; 1.6137x over previous
import functools

import jax
import jax.numpy as jnp
from jax import lax
from jax.experimental import pallas as pl
from jax.experimental.pallas import tpu as pltpu


def _bilstm_body(x_ref, wih_ref, whh_ref, b_ref, out_ref, out_tm,
                 *, seq_len, nb, hidden):
    T, Nb, H = seq_len, nb, hidden
    HH = 2 * H
    G = 8 * H

    xb = x_ref[...].astype(jnp.bfloat16)
    xt = xb.transpose(1, 0, 2).reshape(T * Nb, xb.shape[-1])

    wih = wih_ref[...]
    whh = whh_ref[...]
    b = b_ref[...]

    pre = jnp.dot(xt, wih, preferred_element_type=jnp.float32) + b

    lane = lax.broadcasted_iota(jnp.int32, (Nb, G), 1)
    is_fwd = (lane % HH) < H

    h = jnp.zeros((Nb, HH), jnp.float32)
    c = jnp.zeros((Nb, HH), jnp.float32)

    for t in range(T):
        tb = T - 1 - t
        pre_t = jnp.where(is_fwd,
                          pre[t * Nb:(t + 1) * Nb, :],
                          pre[tb * Nb:(tb + 1) * Nb, :])
        gates = pre_t + jnp.dot(h.astype(jnp.bfloat16), whh,
                                preferred_element_type=jnp.float32)
        s_if = jax.nn.sigmoid(gates[:, 0:2 * HH])
        g_g = jnp.tanh(gates[:, 2 * HH:3 * HH])
        o_g = jax.nn.sigmoid(gates[:, 3 * HH:4 * HH])
        c = s_if[:, HH:2 * HH] * c + s_if[:, 0:HH] * g_g
        h = o_g * jnp.tanh(c)
        out_tm[t * Nb:(t + 1) * Nb, 0:H] = h[:, 0:H]
        out_tm[tb * Nb:(tb + 1) * Nb, H:2 * H] = h[:, H:2 * H]

    out_ref[...] = out_tm[...].reshape(T, Nb, HH).transpose(1, 0, 2)


@jax.jit
def kernel(x, w_ih_fused, w_hh_blk, b_fused):
    N, T, E = x.shape
    H = w_hh_blk.shape[0] // 2
    NC = 2
    Nb = N // NC

    wih = w_ih_fused.astype(jnp.bfloat16)
    whh = w_hh_blk.astype(jnp.bfloat16)

    body = functools.partial(_bilstm_body, seq_len=T, nb=Nb, hidden=H)
    out = pl.pallas_call(
        body,
        out_shape=jax.ShapeDtypeStruct((N, T, 2 * H), x.dtype),
        grid=(NC,),
        in_specs=[
            pl.BlockSpec((Nb, T, E), lambda i: (i, 0, 0)),
            pl.BlockSpec((E, 8 * H), lambda i: (0, 0)),
            pl.BlockSpec((2 * H, 8 * H), lambda i: (0, 0)),
            pl.BlockSpec((1, 8 * H), lambda i: (0, 0)),
        ],
        out_specs=pl.BlockSpec((Nb, T, 2 * H), lambda i: (i, 0, 0)),
        scratch_shapes=[pltpu.VMEM((T * Nb, 2 * H), jnp.float32)],
        compiler_params=pltpu.CompilerParams(
            dimension_semantics=("parallel",)),
    )(x, wih, whh, b_fused)

    return out
